# baseline (speedup 1.0000x reference)
import functools

import jax
import jax.numpy as jnp
from jax import lax
from jax.experimental import pallas as pl
from jax.experimental.pallas import tpu as pltpu
from jax.experimental.pallas import tpu_sc as plsc

_NW = 32
_LANES = 16
_CHUNK_ROWS = 40
_CHUNK_COLS = 128
_SINGLE_MODE_CHUNKS = 1000


@functools.partial(jax.jit, static_argnums=(0, 1))
def _lut_gather(n_rows, n_cols, codes, lut32, pair_lut):
    cols_per_w = n_cols // _NW
    n_row_chunks = n_rows // _CHUNK_ROWS
    n_col_chunks = cols_per_w // _CHUNK_COLS
    n_chunks = n_row_chunks * n_col_chunks
    n_pairs = _CHUNK_COLS // (2 * _LANES)
    mesh = plsc.VectorSubcoreMesh(core_axis_name="c", subcore_axis_name="s")

    buf = pltpu.VMEM((_CHUNK_ROWS, _CHUNK_COLS), jnp.int32)

    @functools.partial(
        pl.kernel,
        out_type=jax.ShapeDtypeStruct((n_rows, n_cols), jnp.int32),
        mesh=mesh,
        compiler_params=pltpu.CompilerParams(
            needs_layout_passes=False, use_tc_tiling_on_sc=True),
        scratch_types=(
            [pltpu.VMEM((256,), jnp.int32),
             pltpu.VMEM((65536,), jnp.int32)]
            + [buf] * 4
            + [pltpu.SemaphoreType.DMA] * 5
        ),
    )
    def k(codes_hbm, lut_hbm, pair_hbm, out_hbm, lut_v, pair_v,
          in_v0, in_v1, out_v0, out_v1, isem0, isem1, osem0, osem1, tsem):
        in_bufs = (in_v0, in_v1)
        out_bufs = (out_v0, out_v1)
        isems = (isem0, isem1)
        osems = (osem0, osem1)
        wid = lax.axis_index("s") * 2 + lax.axis_index("c")
        base_col = wid * cols_per_w
        pltpu.sync_copy(lut_hbm, lut_v)
        table_cp = pltpu.async_copy(pair_hbm, pair_v, tsem)
        in_cps = [None, None]
        out_cps = [None, None]

        def chunk_origin(g):
            r = (g % n_row_chunks) * _CHUNK_ROWS
            c = base_col + (g // n_row_chunks) * _CHUNK_COLS
            return r, c

        def start_in(g):
            b = g % 2
            r, c = chunk_origin(g)
            in_cps[b] = pltpu.async_copy(
                codes_hbm.at[pl.ds(r, _CHUNK_ROWS), pl.ds(c, _CHUNK_COLS)],
                in_bufs[b], isems[b])

        start_in(0)
        start_in(1)
        for g in range(n_chunks):
            b = g % 2
            in_cps[b].wait()
            if out_cps[b] is not None:
                out_cps[b].wait()
            if g == _SINGLE_MODE_CHUNKS:
                table_cp.wait()
            in_v, out_v = in_bufs[b], out_bufs[b]

            if g < _SINGLE_MODE_CHUNKS:
                @plsc.parallel_loop(0, _CHUNK_ROWS, 1, unroll=4)
                def body(p):
                    for j in range(_CHUNK_COLS // _LANES):
                        idx = in_v[p, pl.ds(j * _LANES, _LANES)]
                        out_v[p, pl.ds(j * _LANES, _LANES)] = (
                            plsc.load_gather(lut_v, [idx]))
            else:
                @plsc.parallel_loop(0, _CHUNK_ROWS, 1, unroll=4)
                def body(p):
                    for j in range(n_pairs):
                        o = j * 2 * _LANES
                        a = in_v[p, pl.ds(o, _LANES)]
                        bb = in_v[p, pl.ds(o + _LANES, _LANES)]
                        r = plsc.load_gather(
                            pair_v, [a | (bb << jnp.int32(8))])
                        out_v[p, pl.ds(o, _LANES)] = r & jnp.int32(0xFF)
                        out_v[p, pl.ds(o + _LANES, _LANES)] = (
                            r >> jnp.int32(8))

            r, c = chunk_origin(g)
            out_cps[b] = pltpu.async_copy(
                out_bufs[b],
                out_hbm.at[pl.ds(r, _CHUNK_ROWS), pl.ds(c, _CHUNK_COLS)],
                osems[b])
            if g + 2 < n_chunks:
                start_in(g + 2)

        if n_chunks <= _SINGLE_MODE_CHUNKS:
            table_cp.wait()
        for b in range(2):
            if out_cps[b] is not None:
                out_cps[b].wait()

    return k(codes, lut32, pair_lut)


def kernel(char_bytes, lut):
    B, L = char_bytes.shape
    lut32 = lut.astype(jnp.int32)
    pair_lut = ((lut32[:, None] << 8) | lut32[None, :]).reshape(65536)
    codes_t = char_bytes.astype(jnp.int32).T
    out_t = _lut_gather(L, B, codes_t, lut32, pair_lut)
    return out_t.T.astype(lut.dtype)


# speedup vs baseline: 1.4780x; 1.4780x over previous
import functools

import jax
import jax.numpy as jnp
from jax import lax
from jax.experimental import pallas as pl
from jax.experimental.pallas import tpu as pltpu
from jax.experimental.pallas import tpu_sc as plsc

_NW = 32
_LANES = 16
_COLS_PER_CHUNK = 128


@functools.partial(jax.jit, static_argnums=(0, 1))
def _lut_gather(n_rows, n_cols, codes, lut32):
    cols_per_w = n_cols // _NW
    n_chunks = cols_per_w // _COLS_PER_CHUNK
    n_j = _COLS_PER_CHUNK // _LANES
    mesh = plsc.VectorSubcoreMesh(core_axis_name="c", subcore_axis_name="s")

    @functools.partial(
        pl.kernel,
        out_type=jax.ShapeDtypeStruct((n_rows, n_cols), jnp.int32),
        mesh=mesh,
        compiler_params=pltpu.CompilerParams(
            needs_layout_passes=False, use_tc_tiling_on_sc=True,
            disable_bounds_checks=True),
        scratch_types=[
            pltpu.VMEM((256,), jnp.int32),
            pltpu.VMEM((n_rows, _COLS_PER_CHUNK), jnp.int32),
            pltpu.VMEM((n_rows, _COLS_PER_CHUNK), jnp.int32),
            pltpu.VMEM((n_rows, _COLS_PER_CHUNK), jnp.int32),
            pltpu.VMEM((n_rows, _COLS_PER_CHUNK), jnp.int32),
            pltpu.SemaphoreType.DMA,
            pltpu.SemaphoreType.DMA,
            pltpu.SemaphoreType.DMA,
            pltpu.SemaphoreType.DMA,
        ],
    )
    def k(codes_hbm, lut_hbm, out_hbm, lut_v, in_v0, in_v1, out_v0, out_v1,
          isem0, isem1, osem0, osem1):
        wid = lax.axis_index("s") * 2 + lax.axis_index("c")
        base_col = wid * cols_per_w
        pltpu.sync_copy(lut_hbm, lut_v)
        in_bufs = (in_v0, in_v1)
        out_bufs = (out_v0, out_v1)
        isems = (isem0, isem1)
        osems = (osem0, osem1)
        in_cps = [None, None]
        out_cps = [None, None]

        def start_in(g):
            b = g % 2
            in_cps[b] = pltpu.async_copy(
                codes_hbm.at[:, pl.ds(base_col + g * _COLS_PER_CHUNK,
                                      _COLS_PER_CHUNK)],
                in_bufs[b], isems[b])

        start_in(0)
        for g in range(n_chunks):
            b = g % 2
            if g + 1 < n_chunks:
                start_in(g + 1)
            in_cps[b].wait()
            if out_cps[b] is not None:
                out_cps[b].wait()
            in_v, out_v = in_bufs[b], out_bufs[b]

            @plsc.parallel_loop(0, n_rows, 1, unroll=2)
            def body(p):
                for j in range(n_j):
                    idx = in_v[p, pl.ds(j * _LANES, _LANES)]
                    out_v[p, pl.ds(j * _LANES, _LANES)] = plsc.load_gather(
                        lut_v, [idx])

            out_cps[b] = pltpu.async_copy(
                out_bufs[b],
                out_hbm.at[:, pl.ds(base_col + g * _COLS_PER_CHUNK,
                                    _COLS_PER_CHUNK)], osems[b])

        for b in range(2):
            if out_cps[b] is not None:
                out_cps[b].wait()

    return k(codes, lut32)


def kernel(char_bytes, lut):
    B, L = char_bytes.shape
    lut32 = lut.astype(jnp.int32)
    codes_t = char_bytes.astype(jnp.int32).T
    out_t = _lut_gather(L, B, codes_t, lut32)
    return out_t.T.astype(lut.dtype)
